# baseline (speedup 1.0000x reference)
import functools

import jax
import jax.numpy as jnp
from jax import lax
from jax.experimental import pallas as pl
from jax.experimental.pallas import tpu as pltpu
from jax.experimental.pallas import tpu_sc as plsc

_EPS = 1e-12
_LANES = 128
_G = 8
_TOK = 4096


_VB = 32768


def _tc_compact_table(table_t, d):
    v = table_t.shape[1]
    grid = (v + _VB - 1) // _VB
    nrow = -(-v // 8)

    place = jnp.stack(
        [jnp.pad(jnp.eye(d, dtype=jnp.float32), ((0, 0), (d * s, 8 * d - d * (s + 1))))
         for s in range(8)]
    )

    nch = 64
    cw = _VB // nch

    def body(t_ref, p_ref, o_ref, z_ref):
        for c in range(nch):
            z_ref[c * cw:(c + 1) * cw, :] = jnp.swapaxes(
                t_ref[:, c * cw:(c + 1) * cw], 0, 1
            )
            acc = jnp.dot(
                z_ref[c * cw:(c + 1) * cw:8, :], p_ref[0],
                preferred_element_type=jnp.float32,
            )
            for s in range(1, 8):
                acc = acc + jnp.dot(
                    z_ref[c * cw + s:(c + 1) * cw:8, :], p_ref[s],
                    preferred_element_type=jnp.float32,
                )
            o_ref[c * cw // 8:(c + 1) * cw // 8, :] = acc

    return pl.pallas_call(
        body,
        grid=(grid,),
        in_specs=[
            pl.BlockSpec((d, _VB), lambda g: (0, g)),
            pl.BlockSpec((8, d, 8 * d), lambda g: (0, 0, 0)),
        ],
        out_specs=pl.BlockSpec((_VB // 8, 8 * d), lambda g: (g, 0)),
        out_shape=jax.ShapeDtypeStruct((nrow, 8 * d), jnp.float32),
        scratch_shapes=[pltpu.VMEM((_VB, d), jnp.float32)],
    )(table_t, place)


def _sc_gather(table, idx2d):
    nrows, lanes = idx2d.shape
    d = table.shape[1]
    nw = 32
    rpw = nrows // nw
    mesh = plsc.VectorSubcoreMesh(core_axis_name="c", subcore_axis_name="s")

    g = _G
    while rpw % g:
        g -= 1

    @functools.partial(
        pl.kernel,
        mesh=mesh,
        compiler_params=pltpu.CompilerParams(use_tc_tiling_on_sc=False),
        out_type=jax.ShapeDtypeStruct((nrows, lanes, d), jnp.float32),
        scratch_types=[
            pltpu.VMEM((g, lanes), jnp.int32),
            pltpu.VMEM((g, lanes, d), jnp.float32),
            pltpu.SemaphoreType.DMA,
        ],
    )
    def k(table_hbm, idx_hbm, out_hbm, idx_v, rows_v, sem):
        wid = lax.axis_index("s") * 2 + lax.axis_index("c")
        base = wid * rpw

        def body(t, carry):
            row0 = base + t * g
            pltpu.sync_copy(idx_hbm.at[pl.ds(row0, g)], idx_v)
            descs = [
                pltpu.async_copy(table_hbm.at[idx_v.at[j]], rows_v.at[j], sem)
                for j in range(g)
            ]
            for dsc in descs:
                dsc.wait()
            pltpu.sync_copy(rows_v, out_hbm.at[pl.ds(row0, g)])
            return carry

        lax.fori_loop(0, rpw // g, body, 0)

    return k(table, idx2d)


def _tc_proj_ln(e_pack, w2big, b2, gamma, beta, kdim, total_n, blk0, out_prev):
    npack = e_pack.shape[0]
    n_local = npack * 8
    h = w2big.shape[2]
    spp = 128 // kdim

    def body(e_ref, w_ref, b_ref, g_ref, be_ref, *rest):
        o_ref = rest[-1]
        x = e_ref[...]
        for a in range(spp):
            acc = jnp.dot(x, w_ref[a], preferred_element_type=jnp.float32)
            acc = acc + b_ref[...]
            mean = jnp.mean(acc, axis=-1, keepdims=True)
            cen = acc - mean
            var = jnp.mean(cen * cen, axis=-1, keepdims=True)
            y = g_ref[...] * (cen * lax.rsqrt(var + _EPS)) + be_ref[...]
            o_ref[a::spp, :] = y

    in_specs = [
        pl.BlockSpec((_TOK // spp, spp * kdim), lambda i: (i, 0)),
        pl.BlockSpec((spp, spp * kdim, h), lambda i: (0, 0, 0)),
        pl.BlockSpec((1, h), lambda i: (0, 0)),
        pl.BlockSpec((1, h), lambda i: (0, 0)),
        pl.BlockSpec((1, h), lambda i: (0, 0)),
    ]
    args = [e_pack, w2big, b2.reshape(1, h), gamma.reshape(1, h), beta.reshape(1, h)]
    aliases = {}
    if out_prev is not None:
        in_specs.append(pl.BlockSpec(memory_space=pl.ANY))
        args.append(out_prev)
        aliases = {5: 0}
    return pl.pallas_call(
        body,
        grid=(n_local // _TOK,),
        in_specs=in_specs,
        out_specs=pl.BlockSpec((_TOK, h), lambda i: (i + blk0, 0)),
        out_shape=jax.ShapeDtypeStruct((total_n, h), jnp.float32),
        input_output_aliases=aliases,
    )(*args)


def kernel(x, tok_embed1, W2, b2, gamma, beta):
    b, l = x.shape
    v, d = tok_embed1.shape
    hidden = W2.shape[1]
    spp = 128 // d
    idx2d = x.reshape(-1, _LANES)
    table_lin = _tc_compact_table(tok_embed1.T, d).reshape(v, d)
    w2big = jnp.stack(
        [jnp.pad(W2, ((d * a, spp * d - d * (a + 1)), (0, 0))) for a in range(spp)]
    )
    nparts = 8
    prows = idx2d.shape[0] // nparts
    total_n = idx2d.shape[0] * _LANES
    e_parts = [
        _sc_gather(table_lin, idx2d[p * prows:(p + 1) * prows])
        for p in range(nparts)
    ]
    blk_per_part = prows * _LANES // _TOK
    out = None
    for p in range(nparts):
        out = _tc_proj_ln(
            e_parts[p].reshape(-1, spp * d), w2big, b2, gamma, beta, d,
            total_n, p * blk_per_part, out,
        )
    return out.reshape(b, l, hidden)


# speedup vs baseline: 1.0015x; 1.0015x over previous
import functools

import jax
import jax.numpy as jnp
from jax import lax
from jax.experimental import pallas as pl
from jax.experimental.pallas import tpu as pltpu
from jax.experimental.pallas import tpu_sc as plsc

_EPS = 1e-12
_LANES = 128
_G = 8
_TOK = 4096


_VB = 32768


def _tc_compact_table(table_t, d):
    v = table_t.shape[1]
    grid = (v + _VB - 1) // _VB
    nrow = -(-v // 8)

    place = jnp.stack(
        [jnp.pad(jnp.eye(d, dtype=jnp.float32), ((0, 0), (d * s, 8 * d - d * (s + 1))))
         for s in range(8)]
    )

    nch = 64
    cw = _VB // nch

    def body(t_ref, p_ref, o_ref, z_ref):
        for c in range(nch):
            z_ref[c * cw:(c + 1) * cw, :] = jnp.swapaxes(
                t_ref[:, c * cw:(c + 1) * cw], 0, 1
            )
            acc = jnp.dot(
                z_ref[c * cw:(c + 1) * cw:8, :], p_ref[0],
                preferred_element_type=jnp.float32,
            )
            for s in range(1, 8):
                acc = acc + jnp.dot(
                    z_ref[c * cw + s:(c + 1) * cw:8, :], p_ref[s],
                    preferred_element_type=jnp.float32,
                )
            o_ref[c * cw // 8:(c + 1) * cw // 8, :] = acc

    return pl.pallas_call(
        body,
        grid=(grid,),
        in_specs=[
            pl.BlockSpec((d, _VB), lambda g: (0, g)),
            pl.BlockSpec((8, d, 8 * d), lambda g: (0, 0, 0)),
        ],
        out_specs=pl.BlockSpec((_VB // 8, 8 * d), lambda g: (g, 0)),
        out_shape=jax.ShapeDtypeStruct((nrow, 8 * d), jnp.float32),
        scratch_shapes=[pltpu.VMEM((_VB, d), jnp.float32)],
    )(table_t, place)


def _sc_gather(table, idx2d):
    nrows, lanes = idx2d.shape
    d = table.shape[1]
    nw = 32
    rpw = nrows // nw
    mesh = plsc.VectorSubcoreMesh(core_axis_name="c", subcore_axis_name="s")

    g = _G
    while rpw % g:
        g -= 1

    @functools.partial(
        pl.kernel,
        mesh=mesh,
        compiler_params=pltpu.CompilerParams(use_tc_tiling_on_sc=False),
        out_type=jax.ShapeDtypeStruct((nrows, lanes, d), jnp.float32),
        scratch_types=[
            pltpu.VMEM((g, lanes), jnp.int32),
            pltpu.VMEM((g, lanes, d), jnp.float32),
            pltpu.SemaphoreType.DMA,
        ],
    )
    def k(table_hbm, idx_hbm, out_hbm, idx_v, rows_v, sem):
        wid = lax.axis_index("s") * 2 + lax.axis_index("c")
        base = wid * rpw

        def body(t, carry):
            row0 = base + t * g
            pltpu.sync_copy(idx_hbm.at[pl.ds(row0, g)], idx_v)
            descs = [
                pltpu.async_copy(table_hbm.at[idx_v.at[j]], rows_v.at[j], sem)
                for j in range(g)
            ]
            for dsc in descs:
                dsc.wait()
            pltpu.sync_copy(rows_v, out_hbm.at[pl.ds(row0, g)])
            return carry

        lax.fori_loop(0, rpw // g, body, 0)

    return k(table, idx2d)


def _tc_proj_ln(e_pack, w2big, b2, gamma, beta, kdim, total_n, blk0, out_prev):
    npack = e_pack.shape[0]
    n_local = npack * 8
    h = w2big.shape[2]
    spp = 128 // kdim

    def body(e_ref, w_ref, b_ref, g_ref, be_ref, *rest):
        o_ref = rest[-1]
        x = e_ref[...]
        for a in range(spp):
            acc = jnp.dot(x, w_ref[a], preferred_element_type=jnp.float32)
            acc = acc + b_ref[...]
            mean = jnp.mean(acc, axis=-1, keepdims=True)
            cen = acc - mean
            var = jnp.mean(cen * cen, axis=-1, keepdims=True)
            y = g_ref[...] * (cen * lax.rsqrt(var + _EPS)) + be_ref[...]
            o_ref[a::spp, :] = y

    in_specs = [
        pl.BlockSpec((_TOK // spp, spp * kdim), lambda i: (i, 0)),
        pl.BlockSpec((spp, spp * kdim, h), lambda i: (0, 0, 0)),
        pl.BlockSpec((1, h), lambda i: (0, 0)),
        pl.BlockSpec((1, h), lambda i: (0, 0)),
        pl.BlockSpec((1, h), lambda i: (0, 0)),
    ]
    args = [e_pack, w2big, b2.reshape(1, h), gamma.reshape(1, h), beta.reshape(1, h)]
    aliases = {}
    if out_prev is not None:
        in_specs.append(pl.BlockSpec(memory_space=pl.ANY))
        args.append(out_prev)
        aliases = {5: 0}
    return pl.pallas_call(
        body,
        grid=(n_local // _TOK,),
        in_specs=in_specs,
        out_specs=pl.BlockSpec((_TOK, h), lambda i: (i + blk0, 0)),
        out_shape=jax.ShapeDtypeStruct((total_n, h), jnp.float32),
        input_output_aliases=aliases,
    )(*args)


def kernel(x, tok_embed1, W2, b2, gamma, beta):
    b, l = x.shape
    v, d = tok_embed1.shape
    hidden = W2.shape[1]
    spp = 128 // d
    idx2d = x.reshape(-1, _LANES)
    table_lin = _tc_compact_table(tok_embed1.T, d).reshape(v, d)
    w2big = jnp.stack(
        [jnp.pad(W2, ((d * a, spp * d - d * (a + 1)), (0, 0))) for a in range(spp)]
    )
    nparts = 4
    prows = idx2d.shape[0] // nparts
    total_n = idx2d.shape[0] * _LANES
    e_parts = [
        _sc_gather(table_lin, idx2d[p * prows:(p + 1) * prows])
        for p in range(nparts)
    ]
    blk_per_part = prows * _LANES // _TOK
    out = None
    for p in range(nparts):
        out = _tc_proj_ln(
            e_parts[p].reshape(-1, spp * d), w2big, b2, gamma, beta, d,
            total_n, p * blk_per_part, out,
        )
    return out.reshape(b, l, hidden)


# speedup vs baseline: 1.0354x; 1.0338x over previous
import functools

import jax
import jax.numpy as jnp
from jax import lax
from jax.experimental import pallas as pl
from jax.experimental.pallas import tpu as pltpu
from jax.experimental.pallas import tpu_sc as plsc

_EPS = 1e-12
_LANES = 128
_G = 8
_TOK = 8192


_VB = 32768


def _tc_compact_table(table_t, d):
    v = table_t.shape[1]
    grid = (v + _VB - 1) // _VB
    nrow = -(-v // 8)

    place = jnp.stack(
        [jnp.pad(jnp.eye(d, dtype=jnp.float32), ((0, 0), (d * s, 8 * d - d * (s + 1))))
         for s in range(8)]
    )

    nch = 64
    cw = _VB // nch

    def body(t_ref, p_ref, o_ref, z_ref):
        for c in range(nch):
            z_ref[c * cw:(c + 1) * cw, :] = jnp.swapaxes(
                t_ref[:, c * cw:(c + 1) * cw], 0, 1
            )
            acc = jnp.dot(
                z_ref[c * cw:(c + 1) * cw:8, :], p_ref[0],
                preferred_element_type=jnp.float32,
            )
            for s in range(1, 8):
                acc = acc + jnp.dot(
                    z_ref[c * cw + s:(c + 1) * cw:8, :], p_ref[s],
                    preferred_element_type=jnp.float32,
                )
            o_ref[c * cw // 8:(c + 1) * cw // 8, :] = acc

    return pl.pallas_call(
        body,
        grid=(grid,),
        in_specs=[
            pl.BlockSpec((d, _VB), lambda g: (0, g)),
            pl.BlockSpec((8, d, 8 * d), lambda g: (0, 0, 0)),
        ],
        out_specs=pl.BlockSpec((_VB // 8, 8 * d), lambda g: (g, 0)),
        out_shape=jax.ShapeDtypeStruct((nrow, 8 * d), jnp.float32),
        scratch_shapes=[pltpu.VMEM((_VB, d), jnp.float32)],
    )(table_t, place)


def _sc_gather(table, idx2d):
    nrows, lanes = idx2d.shape
    d = table.shape[1]
    nw = 32
    rpw = nrows // nw
    mesh = plsc.VectorSubcoreMesh(core_axis_name="c", subcore_axis_name="s")

    g = _G
    while rpw % g:
        g -= 1

    @functools.partial(
        pl.kernel,
        mesh=mesh,
        compiler_params=pltpu.CompilerParams(use_tc_tiling_on_sc=False),
        out_type=jax.ShapeDtypeStruct((nrows, lanes, d), jnp.float32),
        scratch_types=[
            pltpu.VMEM((g, lanes), jnp.int32),
            pltpu.VMEM((g, lanes, d), jnp.float32),
            pltpu.SemaphoreType.DMA,
        ],
    )
    def k(table_hbm, idx_hbm, out_hbm, idx_v, rows_v, sem):
        wid = lax.axis_index("s") * 2 + lax.axis_index("c")
        base = wid * rpw

        def body(t, carry):
            row0 = base + t * g
            pltpu.sync_copy(idx_hbm.at[pl.ds(row0, g)], idx_v)
            descs = [
                pltpu.async_copy(table_hbm.at[idx_v.at[j]], rows_v.at[j], sem)
                for j in range(g)
            ]
            for dsc in descs:
                dsc.wait()
            pltpu.sync_copy(rows_v, out_hbm.at[pl.ds(row0, g)])
            return carry

        lax.fori_loop(0, rpw // g, body, 0)

    return k(table, idx2d)


def _tc_proj_ln(e_pack, w2big, b2, gamma, beta, kdim, total_n, blk0, out_prev):
    npack = e_pack.shape[0]
    n_local = npack * 8
    h = w2big.shape[2]
    spp = 128 // kdim

    def body(e_ref, w_ref, b_ref, g_ref, be_ref, *rest):
        o_ref = rest[-1]
        x = e_ref[...]
        for a in range(spp):
            acc = jnp.dot(x, w_ref[a], preferred_element_type=jnp.float32)
            acc = acc + b_ref[...]
            mean = jnp.mean(acc, axis=-1, keepdims=True)
            cen = acc - mean
            var = jnp.mean(cen * cen, axis=-1, keepdims=True)
            y = g_ref[...] * (cen * lax.rsqrt(var + _EPS)) + be_ref[...]
            o_ref[a::spp, :] = y

    in_specs = [
        pl.BlockSpec((_TOK // spp, spp * kdim), lambda i: (i, 0)),
        pl.BlockSpec((spp, spp * kdim, h), lambda i: (0, 0, 0)),
        pl.BlockSpec((1, h), lambda i: (0, 0)),
        pl.BlockSpec((1, h), lambda i: (0, 0)),
        pl.BlockSpec((1, h), lambda i: (0, 0)),
    ]
    args = [e_pack, w2big, b2.reshape(1, h), gamma.reshape(1, h), beta.reshape(1, h)]
    aliases = {}
    if out_prev is not None:
        in_specs.append(pl.BlockSpec(memory_space=pl.ANY))
        args.append(out_prev)
        aliases = {5: 0}
    return pl.pallas_call(
        body,
        grid=(n_local // _TOK,),
        in_specs=in_specs,
        out_specs=pl.BlockSpec((_TOK, h), lambda i: (i + blk0, 0)),
        out_shape=jax.ShapeDtypeStruct((total_n, h), jnp.float32),
        input_output_aliases=aliases,
    )(*args)


def kernel(x, tok_embed1, W2, b2, gamma, beta):
    b, l = x.shape
    v, d = tok_embed1.shape
    hidden = W2.shape[1]
    spp = 128 // d
    idx2d = x.reshape(-1, _LANES)
    table_lin = _tc_compact_table(tok_embed1.T, d).reshape(v, d)
    w2big = jnp.stack(
        [jnp.pad(W2, ((d * a, spp * d - d * (a + 1)), (0, 0))) for a in range(spp)]
    )
    nparts = 4
    prows = idx2d.shape[0] // nparts
    total_n = idx2d.shape[0] * _LANES
    e_parts = [
        _sc_gather(table_lin, idx2d[p * prows:(p + 1) * prows])
        for p in range(nparts)
    ]
    blk_per_part = prows * _LANES // _TOK
    out = None
    for p in range(nparts):
        out = _tc_proj_ln(
            e_parts[p].reshape(-1, spp * d), w2big, b2, gamma, beta, d,
            total_n, p * blk_per_part, out,
        )
    return out.reshape(b, l, hidden)
